# baseline (speedup 1.0000x reference)
import functools

import numpy as np
import jax
import jax.numpy as jnp
from jax import lax
from jax.experimental import pallas as pl
from jax.experimental.pallas import tpu as pltpu
from jax.experimental.pallas import tpu_sc as plsc

_NUM_FIELDS = 26
_DIM = 16
_B = 16384
_EIN = _NUM_FIELDS * _DIM
_N_IDX = _B * _NUM_FIELDS
_FIELD_SIZE = 100000
_BN_INV = float(1.0 / np.sqrt(1.0 + 1e-5))

_NW = 32
_PER_W = _N_IDX // _NW
_CHUNK = 3328
_NCHUNKS = _PER_W // _CHUNK

_TCH = 1024
_V = 2600000
_NTCH = _V // _TCH
_TAIL = _V - _NTCH * _TCH
_TAIL_W = _NTCH % _NW


_TR_R = 8192


def _tc_transpose(embT):
    v = embT.shape[1]
    grid = v // _TR_R

    def body(in_ref, out_ref):
        x = in_ref[...]
        eye = jnp.eye(_DIM, dtype=jnp.float32)
        t = lax.dot_general(x, eye, (((0,), (0,)), ((), ())),
                            preferred_element_type=jnp.float32)
        t = t.reshape(_TR_R // 8, 8, _DIM)
        out_ref[...] = jnp.concatenate(
            [t[:, s, :] for s in range(8)], axis=1)

    n_blk = (v + _TR_R - 1) // _TR_R
    return pl.pallas_call(
        body,
        grid=(n_blk,),
        in_specs=[pl.BlockSpec((_DIM, _TR_R), lambda i: (0, i))],
        out_specs=pl.BlockSpec((_TR_R // 8, 8 * _DIM), lambda i: (i, 0)),
        out_shape=jax.ShapeDtypeStruct((v // 8, 8 * _DIM), jnp.float32),
    )(embT)


def _sc_gather(emb_lin, lin_flat, xi):
    mesh = plsc.VectorSubcoreMesh(core_axis_name="c", subcore_axis_name="s")

    @functools.partial(
        pl.kernel,
        mesh=mesh,
        compiler_params=pltpu.CompilerParams(use_tc_tiling_on_sc=False),
        out_type=(
            jax.ShapeDtypeStruct((_N_IDX, _DIM), jnp.float32),
            jax.ShapeDtypeStruct((_N_IDX, 1), jnp.float32),
        ),
        scratch_types=[
            pltpu.VMEM((_CHUNK,), jnp.int32),
            pltpu.VMEM((_CHUNK, _DIM), jnp.float32),
            pltpu.VMEM((_CHUNK, 1), jnp.float32),
            pltpu.SemaphoreType.DMA,
            pltpu.SemaphoreType.DMA,
        ],
    )
    def gather_kernel(emb_hbm, lin_hbm, idx_hbm, e_out, l_out,
                      idx_v, rows_v, lrows_v, sem_e, sem_l):
        wid = lax.axis_index("s") * 2 + lax.axis_index("c")
        base = wid * _PER_W
        for j in range(_NCHUNKS):
            off = base + j * _CHUNK
            pltpu.sync_copy(idx_hbm.at[pl.ds(off, _CHUNK)], idx_v)
            cp_e = pltpu.async_copy(emb_hbm.at[idx_v], rows_v, sem_e)
            cp_l = pltpu.async_copy(lin_hbm.at[idx_v], lrows_v, sem_l)
            cp_e.wait()
            cp_l.wait()
            pltpu.sync_copy(rows_v, e_out.at[pl.ds(off, _CHUNK)])
            pltpu.sync_copy(lrows_v, l_out.at[pl.ds(off, _CHUNK)])

    return gather_kernel(emb_lin, lin_flat, xi)


def _tc_body(e_ref, lv_ref, w1c_ref, b1_ref, g1_ref, be1_ref,
             w2_ref, b2_ref, g2_ref, be2_ref, w3_ref, b3_ref, lb_ref,
             o_ref):
    e = e_ref[...]
    h1s = jnp.dot(e, w1c_ref[...], preferred_element_type=jnp.float32)
    h1 = h1s[:, :_DIM]
    s = h1s[:, _DIM:]
    fm = 0.5 * (jnp.sum(s * s, axis=1) - jnp.sum(e * e, axis=1))
    linear = jnp.sum(lv_ref[...], axis=1) + lb_ref[0, 0]
    h = (h1 + b1_ref[...]) * (g1_ref[...] * _BN_INV) + be1_ref[...]
    h = jnp.maximum(h, 0.0)
    h = jnp.dot(h, w2_ref[...], preferred_element_type=jnp.float32)
    h = (h + b2_ref[...]) * (g2_ref[...] * _BN_INV) + be2_ref[...]
    h = jnp.maximum(h, 0.0)
    mlp = jnp.dot(h, w3_ref[...], preferred_element_type=jnp.float32)[:, 0]
    mlp = mlp + b3_ref[0, 0]
    o_ref[...] = linear + fm + mlp


def _tc_compute(e2d, linv, w1c, b1, g1, be1, w2, b2, g2, be2, w3, b3, lin_b):
    bs = 512
    nblk = _B // bs
    full = lambda shape: pl.BlockSpec(shape, lambda i: (0, 0))
    out2d = pl.pallas_call(
        _tc_body,
        grid=(nblk,),
        in_specs=[
            pl.BlockSpec((bs, _EIN), lambda i: (i, 0)),
            pl.BlockSpec((bs, _NUM_FIELDS), lambda i: (i, 0)),
            full((_EIN, 2 * _DIM)),
            full((1, _DIM)), full((1, _DIM)), full((1, _DIM)),
            full((_DIM, _DIM)),
            full((1, _DIM)), full((1, _DIM)), full((1, _DIM)),
            full((_DIM, 1)), full((1, 1)), full((1, 1)),
        ],
        out_specs=pl.BlockSpec((bs,), lambda i: (i,)),
        out_shape=jax.ShapeDtypeStruct((_B,), jnp.float32),
    )(e2d, linv, w1c, b1, g1, be1, w2, b2, g2, be2, w3, b3, lin_b)
    return out2d


def kernel(x, emb, lin_w, lin_b, W1, b1, g1, be1, W2, b2, g2, be2, W3, b3):
    offsets = jnp.arange(_NUM_FIELDS, dtype=x.dtype) * _FIELD_SIZE
    xi = (x + offsets[None, :]).reshape(_N_IDX).astype(jnp.int32)
    emb_lin = _tc_transpose(emb.T).reshape(emb.shape[0], _DIM)
    e_flat, lin_vals = _sc_gather(emb_lin, lin_w, xi)
    sel = jnp.tile(jnp.eye(_DIM, dtype=jnp.float32), (_NUM_FIELDS, 1))
    w1c = jnp.concatenate([W1, sel], axis=1)
    return _tc_compute(
        e_flat.reshape(_B, _EIN), lin_vals.reshape(_B, _NUM_FIELDS),
        w1c, b1.reshape(1, _DIM), g1.reshape(1, _DIM), be1.reshape(1, _DIM),
        W2, b2.reshape(1, _DIM), g2.reshape(1, _DIM), be2.reshape(1, _DIM),
        W3, b3.reshape(1, 1), lin_b.reshape(1, 1))


# speedup vs baseline: 6.8807x; 6.8807x over previous
import functools

import numpy as np
import jax
import jax.numpy as jnp
from jax import lax
from jax.experimental import pallas as pl
from jax.experimental.pallas import tpu as pltpu
from jax.experimental.pallas import tpu_sc as plsc

_NUM_FIELDS = 26
_DIM = 16
_B = 16384
_EIN = _NUM_FIELDS * _DIM
_N_IDX = _B * _NUM_FIELDS
_FIELD_SIZE = 100000
_BN_INV = float(1.0 / np.sqrt(1.0 + 1e-5))

_NW = 32
_PER_W = _N_IDX // _NW
_CHUNK = 3328
_NCHUNKS = _PER_W // _CHUNK

_TR_R = 16384


def _tc_transpose(embT):
    v = embT.shape[1]

    def body(in_ref, out_ref):
        x = in_ref[...]
        s_all = jnp.tile(jnp.eye(_DIM, dtype=jnp.float32), (1, 8))
        m = lax.dot_general(x, s_all, (((0,), (0,)), ((), ())),
                            preferred_element_type=jnp.float32)
        m3 = m.reshape(_TR_R // 8, 8, 8 * _DIM)
        lane = lax.broadcasted_iota(jnp.int32, (8, 8 * _DIM), 1) // _DIM
        sub = lax.broadcasted_iota(jnp.int32, (8, 8 * _DIM), 0)
        z = (lane == sub).astype(jnp.float32)
        out_ref[...] = jnp.sum(m3 * z[None], axis=1)

    n_blk = (v + _TR_R - 1) // _TR_R
    return pl.pallas_call(
        body,
        grid=(n_blk,),
        in_specs=[pl.BlockSpec((_DIM, _TR_R), lambda i: (0, i))],
        out_specs=pl.BlockSpec((_TR_R // 8, 8 * _DIM), lambda i: (i, 0)),
        out_shape=jax.ShapeDtypeStruct((v // 8, 8 * _DIM), jnp.float32),
    )(embT)


def _sc_gather(emb_lin, lin_flat, xi):
    mesh = plsc.VectorSubcoreMesh(core_axis_name="c", subcore_axis_name="s")

    @functools.partial(
        pl.kernel,
        mesh=mesh,
        compiler_params=pltpu.CompilerParams(use_tc_tiling_on_sc=False),
        out_type=(
            jax.ShapeDtypeStruct((_N_IDX, _DIM), jnp.float32),
            jax.ShapeDtypeStruct((_N_IDX,), jnp.float32),
        ),
        scratch_types=[
            pltpu.VMEM((_CHUNK,), jnp.int32),
            pltpu.VMEM((_CHUNK, _DIM), jnp.float32),
            pltpu.VMEM((_CHUNK,), jnp.float32),
            pltpu.SemaphoreType.DMA,
            pltpu.SemaphoreType.DMA,
        ],
    )
    def gather_kernel(emb_hbm, lin_hbm, idx_hbm, e_out, l_out,
                      idx_v, rows_v, lrows_v, sem_e, sem_l):
        wid = lax.axis_index("s") * 2 + lax.axis_index("c")
        base = wid * _PER_W
        for j in range(_NCHUNKS):
            off = base + j * _CHUNK
            pltpu.sync_copy(idx_hbm.at[pl.ds(off, _CHUNK)], idx_v)
            cp_e = pltpu.async_copy(emb_hbm.at[idx_v], rows_v, sem_e)
            cp_l = pltpu.async_copy(lin_hbm.at[idx_v], lrows_v, sem_l)
            cp_e.wait()
            cp_l.wait()
            pltpu.sync_copy(rows_v, e_out.at[pl.ds(off, _CHUNK)])
            pltpu.sync_copy(lrows_v, l_out.at[pl.ds(off, _CHUNK)])

    return gather_kernel(emb_lin, lin_flat, xi)


def _tc_body(e_ref, lv_ref, w1c_ref, b1_ref, g1_ref, be1_ref,
             w2_ref, b2_ref, g2_ref, be2_ref, w3_ref, b3_ref, lb_ref,
             o_ref):
    e = e_ref[...]
    h1s = jnp.dot(e, w1c_ref[...], preferred_element_type=jnp.float32)
    h1 = h1s[:, :_DIM]
    s = h1s[:, _DIM:]
    fm = 0.5 * (jnp.sum(s * s, axis=1) - jnp.sum(e * e, axis=1))
    linear = jnp.sum(lv_ref[...], axis=1) + lb_ref[0, 0]
    h = (h1 + b1_ref[...]) * (g1_ref[...] * _BN_INV) + be1_ref[...]
    h = jnp.maximum(h, 0.0)
    h = jnp.dot(h, w2_ref[...], preferred_element_type=jnp.float32)
    h = (h + b2_ref[...]) * (g2_ref[...] * _BN_INV) + be2_ref[...]
    h = jnp.maximum(h, 0.0)
    mlp = jnp.dot(h, w3_ref[...], preferred_element_type=jnp.float32)[:, 0]
    mlp = mlp + b3_ref[0, 0]
    o_ref[...] = linear + fm + mlp


def _tc_compute(e2d, linv, w1c, b1, g1, be1, w2, b2, g2, be2, w3, b3, lin_b):
    bs = 512
    nblk = _B // bs
    full = lambda shape: pl.BlockSpec(shape, lambda i: (0, 0))
    out2d = pl.pallas_call(
        _tc_body,
        grid=(nblk,),
        in_specs=[
            pl.BlockSpec((bs, _EIN), lambda i: (i, 0)),
            pl.BlockSpec((bs, _NUM_FIELDS), lambda i: (i, 0)),
            full((_EIN, 2 * _DIM)),
            full((1, _DIM)), full((1, _DIM)), full((1, _DIM)),
            full((_DIM, _DIM)),
            full((1, _DIM)), full((1, _DIM)), full((1, _DIM)),
            full((_DIM, 1)), full((1, 1)), full((1, 1)),
        ],
        out_specs=pl.BlockSpec((bs,), lambda i: (i,)),
        out_shape=jax.ShapeDtypeStruct((_B,), jnp.float32),
    )(e2d, linv, w1c, b1, g1, be1, w2, b2, g2, be2, w3, b3, lin_b)
    return out2d


def kernel(x, emb, lin_w, lin_b, W1, b1, g1, be1, W2, b2, g2, be2, W3, b3):
    offsets = jnp.arange(_NUM_FIELDS, dtype=x.dtype) * _FIELD_SIZE
    xi = (x + offsets[None, :]).reshape(_N_IDX).astype(jnp.int32)
    emb_lin = _tc_transpose(emb.T).reshape(emb.shape[0], _DIM)
    e_flat, lin_vals = _sc_gather(emb_lin, lin_w.T.reshape(-1), xi)
    sel = jnp.tile(jnp.eye(_DIM, dtype=jnp.float32), (_NUM_FIELDS, 1))
    w1c = jnp.concatenate([W1, sel], axis=1)
    return _tc_compute(
        e_flat.reshape(_B, _EIN), lin_vals.reshape(_B, _NUM_FIELDS),
        w1c, b1.reshape(1, _DIM), g1.reshape(1, _DIM), be1.reshape(1, _DIM),
        W2, b2.reshape(1, _DIM), g2.reshape(1, _DIM), be2.reshape(1, _DIM),
        W3, b3.reshape(1, 1), lin_b.reshape(1, 1))
